# baseline (speedup 1.0000x reference)
import jax
import jax.numpy as jnp
from jax.experimental import pallas as pl

_H, _W, _C = 32, 32, 96
_HW = _H * _W
_NCLS = 150
_TAU = 0.3


def _body(prob_ref, emb_ref, out_ref):
  for i in range(16):
    p = prob_ref[i]
    e = emb_ref[i]
    m = jnp.max(p, axis=0, keepdims=True)
    s = jnp.sum(p, axis=0, keepdims=True)
    hw_iota = jax.lax.broadcasted_iota(jnp.int32, p.shape, 0)
    idx = jnp.min(jnp.where(p == m, hw_iota, _HW), axis=0, keepdims=True)
    rep = (s * (1.0 / _HW)) > _TAU
    onehot = ((hw_iota == idx) & rep).astype(jnp.float32)
    out_ref[i] = jax.lax.dot_general(
        onehot, e, (((0,), (0,)), ((), ())),
        preferred_element_type=jnp.float32,
    )


def kernel(emb, prob_map):
    B = emb.shape[0]
    emb_flat = emb.reshape(B, _HW, _C)
    prob_flat = prob_map.reshape(B, _HW, _NCLS)
    out = pl.pallas_call(
        _body,
        grid=(B // 16,),
        in_specs=[
            pl.BlockSpec((16, _HW, _NCLS), lambda b: (b, 0, 0)),
            pl.BlockSpec((16, _HW, _C), lambda b: (b, 0, 0)),
        ],
        out_specs=pl.BlockSpec((16, _NCLS, _C), lambda b: (b, 0, 0)),
        out_shape=jax.ShapeDtypeStruct((B, _NCLS, _C), jnp.float32),
    )(prob_flat, emb_flat)
    return out


# speedup vs baseline: 1.1178x; 1.1178x over previous
import jax
import jax.numpy as jnp
from jax.experimental import pallas as pl

_H, _W, _C = 32, 32, 96
_HW = _H * _W
_NCLS = 150
_TAU = 0.3


def _body(prob_ref, emb_ref, out_ref):
  for i in range(8):
    p = prob_ref[i]
    e = emb_ref[i]
    m = jnp.max(p, axis=0, keepdims=True)
    ones_row = jnp.ones((1, _HW), jnp.float32)
    s = jax.lax.dot_general(ones_row, p, (((1,), (0,)), ((), ())),
                            preferred_element_type=jnp.float32)
    hw_iota = jax.lax.broadcasted_iota(jnp.int32, p.shape, 0)
    idx = jnp.min(jnp.where(p == m, hw_iota, _HW), axis=0, keepdims=True)
    rep = (s * (1.0 / _HW)) > _TAU
    onehot = ((hw_iota == idx) & rep).astype(jnp.float32)
    out_ref[i] = jax.lax.dot_general(
        onehot, e, (((0,), (0,)), ((), ())),
        preferred_element_type=jnp.float32,
    )


def kernel(emb, prob_map):
    B = emb.shape[0]
    emb_flat = emb.reshape(B, _HW, _C)
    prob_flat = prob_map.reshape(B, _HW, _NCLS)
    out = pl.pallas_call(
        _body,
        grid=(B // 8,),
        in_specs=[
            pl.BlockSpec((8, _HW, _NCLS), lambda b: (b, 0, 0)),
            pl.BlockSpec((8, _HW, _C), lambda b: (b, 0, 0)),
        ],
        out_specs=pl.BlockSpec((8, _NCLS, _C), lambda b: (b, 0, 0)),
        out_shape=jax.ShapeDtypeStruct((B, _NCLS, _C), jnp.float32),
    )(prob_flat, emb_flat)
    return out
